# baseline (device time: 25714 ns/iter reference)
import jax
import jax.numpy as jnp
from jax import lax
from jax.experimental import pallas as pl
from jax.experimental.pallas import tpu as pltpu

N_DEV = 32
SLICE = 256 // N_DEV

ISSUE_ORDER = [14, 18, 10, 22, 13, 19, 11, 21, 12, 20, 6, 26, 5, 15, 17,
               27, 2, 30, 3, 29, 9, 23, 4, 28, 7, 25, 16, 8, 24, 1, 31]


def kernel(x, W1, W2):
    m, _ = x.shape
    _, n = W2.shape

    def body(x_ref, w1_ref, w2_ref, out_ref, acc_buf, recv1, red_buf,
             send_sems1, recv_sems1, send_sems2, recv_sems2):
        my_pos = lax.axis_index("i")

        barrier_sem = pltpu.get_barrier_semaphore()
        for d in range(1, N_DEV):
            pl.semaphore_signal(
                barrier_sem, inc=1,
                device_id=((my_pos + d) % N_DEV,),
                device_id_type=pl.DeviceIdType.MESH,
            )

        h = jnp.maximum(
            jnp.dot(x_ref[:, :], w1_ref[:, :],
                    preferred_element_type=jnp.float32),
            0.0,
        )
        acc_buf[:, :] = jnp.dot(h, w2_ref[:, :],
                                preferred_element_type=jnp.float32)

        pl.semaphore_wait(barrier_sem, N_DEV - 1)

        r1 = []
        for d in ISSUE_ORDER:
            t = (my_pos + d) % N_DEV
            rdma = pltpu.make_async_remote_copy(
                src_ref=acc_buf.at[pl.ds(t * SLICE, SLICE)],
                dst_ref=recv1.at[d - 1],
                send_sem=send_sems1.at[d - 1],
                recv_sem=recv_sems1.at[d - 1],
                device_id=(t,),
                device_id_type=pl.DeviceIdType.MESH,
            )
            rdma.start()
            r1.append(rdma)
        for rdma in r1:
            rdma.wait_recv()

        red_buf[:, :] = (
            acc_buf[pl.ds(my_pos * SLICE, SLICE), :]
            + jnp.sum(recv1[:, :, :], axis=0)
        )

        r2 = []
        for d in ISSUE_ORDER:
            t = (my_pos + d) % N_DEV
            rdma = pltpu.make_async_remote_copy(
                src_ref=red_buf,
                dst_ref=out_ref.at[pl.ds(my_pos * SLICE, SLICE)],
                send_sem=send_sems2.at[d - 1],
                recv_sem=recv_sems2.at[d - 1],
                device_id=(t,),
                device_id_type=pl.DeviceIdType.MESH,
            )
            rdma.start()
            r2.append(rdma)
        out_ref[pl.ds(my_pos * SLICE, SLICE), :] = red_buf[:, :]
        for rdma in r2:
            rdma.wait_recv()

        for rdma in r1:
            rdma.wait_send()
        for rdma in r2:
            rdma.wait_send()

    return pl.pallas_call(
        body,
        out_shape=jax.ShapeDtypeStruct((m, n), jnp.float32),
        in_specs=[pl.BlockSpec(memory_space=pltpu.VMEM)] * 3,
        out_specs=pl.BlockSpec(memory_space=pltpu.VMEM),
        scratch_shapes=[
            pltpu.VMEM((m, n), jnp.float32),
            pltpu.VMEM((N_DEV - 1, SLICE, n), jnp.float32),
            pltpu.VMEM((SLICE, n), jnp.float32),
            pltpu.SemaphoreType.DMA((N_DEV - 1,)),
            pltpu.SemaphoreType.DMA((N_DEV - 1,)),
            pltpu.SemaphoreType.DMA((N_DEV - 1,)),
            pltpu.SemaphoreType.DMA((N_DEV - 1,)),
        ],
        compiler_params=pltpu.CompilerParams(collective_id=0),
    )(x, W1, W2)


# device time: 20898 ns/iter; 1.2305x vs baseline; 1.2305x over previous
import jax
import jax.numpy as jnp
from jax import lax
from jax.experimental import pallas as pl
from jax.experimental.pallas import tpu as pltpu

N_DEV = 32
SLICE = 256 // N_DEV

ISSUE_ORDER = [14, 18, 10, 22, 13, 19, 11, 21, 12, 20, 6, 26, 5, 15, 17,
               27, 2, 30, 3, 29, 9, 23, 4, 28, 7, 25, 16, 8, 24, 1, 31]


def kernel(x, W1, W2):
    m, _ = x.shape
    _, n = W2.shape

    def body(x_ref, w1_ref, w2_ref, out_ref, acc_buf, recv1, red_buf,
             send_sems1, recv_sems1, send_sems2, recv_sems2):
        my_pos = lax.axis_index("i")

        barrier_sem = pltpu.get_barrier_semaphore()
        pl.semaphore_signal(
            barrier_sem, inc=1,
            device_id=(my_pos,), device_id_type=pl.DeviceIdType.MESH,
        )

        h = jnp.maximum(
            jnp.dot(x_ref[:, :], w1_ref[:, :],
                    preferred_element_type=jnp.float32),
            0.0,
        )
        acc_buf[:, :] = jnp.dot(h, w2_ref[:, :],
                                preferred_element_type=jnp.float32)

        pl.semaphore_wait(barrier_sem, 1)

        r1 = []
        for d in ISSUE_ORDER:
            t = (my_pos + d) % N_DEV
            rdma = pltpu.make_async_remote_copy(
                src_ref=acc_buf.at[pl.ds(t * SLICE, SLICE)],
                dst_ref=recv1.at[d - 1],
                send_sem=send_sems1.at[d - 1],
                recv_sem=recv_sems1.at[d - 1],
                device_id=(t,),
                device_id_type=pl.DeviceIdType.MESH,
            )
            rdma.start()
            r1.append(rdma)
        for rdma in r1:
            rdma.wait_recv()

        red_buf[:, :] = (
            acc_buf[pl.ds(my_pos * SLICE, SLICE), :]
            + jnp.sum(recv1[:, :, :], axis=0)
        )

        r2 = []
        for d in ISSUE_ORDER:
            t = (my_pos + d) % N_DEV
            rdma = pltpu.make_async_remote_copy(
                src_ref=red_buf,
                dst_ref=out_ref.at[pl.ds(my_pos * SLICE, SLICE)],
                send_sem=send_sems2.at[d - 1],
                recv_sem=recv_sems2.at[d - 1],
                device_id=(t,),
                device_id_type=pl.DeviceIdType.MESH,
            )
            rdma.start()
            r2.append(rdma)
        out_ref[pl.ds(my_pos * SLICE, SLICE), :] = red_buf[:, :]
        for rdma in r2:
            rdma.wait_recv()

        for rdma in r1:
            rdma.wait_send()
        for rdma in r2:
            rdma.wait_send()

    return pl.pallas_call(
        body,
        out_shape=jax.ShapeDtypeStruct((m, n), jnp.float32),
        in_specs=[pl.BlockSpec(memory_space=pltpu.VMEM)] * 3,
        out_specs=pl.BlockSpec(memory_space=pltpu.VMEM),
        scratch_shapes=[
            pltpu.VMEM((m, n), jnp.float32),
            pltpu.VMEM((N_DEV - 1, SLICE, n), jnp.float32),
            pltpu.VMEM((SLICE, n), jnp.float32),
            pltpu.SemaphoreType.DMA((N_DEV - 1,)),
            pltpu.SemaphoreType.DMA((N_DEV - 1,)),
            pltpu.SemaphoreType.DMA((N_DEV - 1,)),
            pltpu.SemaphoreType.DMA((N_DEV - 1,)),
        ],
        compiler_params=pltpu.CompilerParams(collective_id=0),
    )(x, W1, W2)


# device time: 19470 ns/iter; 1.3207x vs baseline; 1.0733x over previous
import jax
import jax.numpy as jnp
from jax import lax
from jax.experimental import pallas as pl
from jax.experimental.pallas import tpu as pltpu

N_DEV = 32
SLICE = 256 // N_DEV

ORDER_H1 = [14, 10, 13, 11, 12, 6, 5, 15, 2, 3, 9, 4, 7, 16, 8, 1]
ORDER_H2 = [18, 22, 19, 21, 20, 26, 17, 27, 30, 29, 23, 28, 25, 24, 31]


def kernel(x, W1, W2):
    m, _ = x.shape
    _, n = W2.shape
    half = m // 2

    def body(x2_ref, w1_ref, w2_ref, out_ref, acc_buf, recv1, red_buf,
             send_sems1, recv_sems1, send_sems2, recv_sems2):
        my_pos = lax.axis_index("i")

        barrier_sem = pltpu.get_barrier_semaphore()
        pl.semaphore_signal(
            barrier_sem, inc=1,
            device_id=(my_pos,), device_id_type=pl.DeviceIdType.MESH,
        )
        pl.semaphore_wait(barrier_sem, 1)

        def send_r1(d):
            t = (my_pos + d) % N_DEV
            rdma = pltpu.make_async_remote_copy(
                src_ref=acc_buf.at[pl.ds((d - 1) * SLICE, SLICE)],
                dst_ref=recv1.at[d - 1],
                send_sem=send_sems1.at[d - 1],
                recv_sem=recv_sems1.at[d - 1],
                device_id=(t,),
                device_id_type=pl.DeviceIdType.MESH,
            )
            rdma.start()
            return rdma

        xr = x2_ref[pl.ds(SLICE * (my_pos + 1), m), :]

        h1 = jnp.maximum(
            jnp.dot(xr[:half, :], w1_ref[:, :],
                    preferred_element_type=jnp.float32),
            0.0,
        )
        acc_buf[:half, :] = jnp.dot(h1, w2_ref[:, :],
                                    preferred_element_type=jnp.float32)
        r1 = [send_r1(d) for d in ORDER_H1]

        h2 = jnp.maximum(
            jnp.dot(xr[half:, :], w1_ref[:, :],
                    preferred_element_type=jnp.float32),
            0.0,
        )
        acc_buf[half:, :] = jnp.dot(h2, w2_ref[:, :],
                                    preferred_element_type=jnp.float32)
        r1 += [send_r1(d) for d in ORDER_H2]

        for rdma in r1:
            rdma.wait_recv()

        red_buf[:, :] = (
            acc_buf[(N_DEV - 1) * SLICE:, :]
            + jnp.sum(recv1[:, :, :], axis=0)
        )

        r2 = []
        for d in ORDER_H1 + ORDER_H2:
            t = (my_pos + d) % N_DEV
            rdma = pltpu.make_async_remote_copy(
                src_ref=red_buf,
                dst_ref=out_ref.at[pl.ds(my_pos * SLICE, SLICE)],
                send_sem=send_sems2.at[d - 1],
                recv_sem=recv_sems2.at[d - 1],
                device_id=(t,),
                device_id_type=pl.DeviceIdType.MESH,
            )
            rdma.start()
            r2.append(rdma)
        out_ref[pl.ds(my_pos * SLICE, SLICE), :] = red_buf[:, :]
        for rdma in r2:
            rdma.wait_recv()

        for rdma in r1:
            rdma.wait_send()
        for rdma in r2:
            rdma.wait_send()

    x2 = jnp.concatenate([x, x], axis=0)
    return pl.pallas_call(
        body,
        out_shape=jax.ShapeDtypeStruct((m, n), jnp.float32),
        in_specs=[pl.BlockSpec(memory_space=pltpu.VMEM)] * 3,
        out_specs=pl.BlockSpec(memory_space=pltpu.VMEM),
        scratch_shapes=[
            pltpu.VMEM((m, n), jnp.float32),
            pltpu.VMEM((N_DEV - 1, SLICE, n), jnp.float32),
            pltpu.VMEM((SLICE, n), jnp.float32),
            pltpu.SemaphoreType.DMA((N_DEV - 1,)),
            pltpu.SemaphoreType.DMA((N_DEV - 1,)),
            pltpu.SemaphoreType.DMA((N_DEV - 1,)),
            pltpu.SemaphoreType.DMA((N_DEV - 1,)),
        ],
        compiler_params=pltpu.CompilerParams(collective_id=0),
    )(x2, W1, W2)
